# Initial kernel scaffold; baseline (speedup 1.0000x reference)
import jax
import jax.numpy as jnp
from jax.experimental import pallas as pl


def kernel(x, edge_index, edge_attr, batch, params):
    raise NotImplementedError("write your pallas kernel here")


# speedup vs baseline: 2.5730x; 2.5730x over previous
import functools

import jax
import jax.numpy as jnp
import numpy as np
from jax import lax
from jax.experimental import pallas as pl
from jax.experimental.pallas import tpu as pltpu
from jax.experimental.pallas import tpu_sc as plsc

D = 128
NW = 32
NC = 2
NSUB = 16
G = 512
CK = 128
NK = 64
BN_EPS = 1e-5

_MESH = plsc.VectorSubcoreMesh(core_axis_name="c", subcore_axis_name="s")


def _wid():
    return lax.axis_index("s") * NC + lax.axis_index("c")


def _zero_vmem(ref, rows, cols):
    zero = jnp.zeros((16,), jnp.float32)

    def zr(r, _):
        def zc(cc, _):
            ref[r, pl.ds(cc * 16, 16)] = zero
            return 0

        return lax.fori_loop(0, cols // 16, zc, 0)

    lax.fori_loop(0, rows, zr, 0)


def _make_prep(NP, ECH, NCH):
    npc = NP // NSUB

    @functools.partial(
        pl.kernel,
        out_type=[
            jax.ShapeDtypeStruct((NP, D), jnp.float32),
            jax.ShapeDtypeStruct((NC, NP, D), jnp.float32),
        ],
        mesh=_MESH,
        scratch_types=[
            pltpu.VMEM((NCH, NK), jnp.int32),
            pltpu.VMEM((ECH, CK), jnp.int32),
            pltpu.VMEM((ECH, CK), jnp.int32),
            pltpu.VMEM((NK, D), jnp.float32),
            pltpu.VMEM((CK, D), jnp.float32),
            pltpu.VMEM_SHARED((NP, D), jnp.float32),
        ],
    )
    def prep(tab_hbm, oh_hbm, idx0_hbm, comb_hbm, dst_hbm, h0_hbm, cpart_hbm,
             idxv, combv, dstv, nrows, rows, c_s):
        c = lax.axis_index("c")
        sid = lax.axis_index("s")
        wid = _wid()
        pltpu.sync_copy(idx0_hbm.at[wid], idxv)
        pltpu.sync_copy(comb_hbm.at[wid], combv)
        pltpu.sync_copy(dst_hbm.at[wid], dstv)

        def gb(j, _):
            pltpu.sync_copy(tab_hbm.at[idxv.at[j]], nrows)
            pltpu.sync_copy(nrows, h0_hbm.at[pl.ds(wid * (NCH * NK) + j * NK, NK)])
            return 0

        lax.fori_loop(0, NCH, gb, 0)

        _zero_vmem(rows, CK, D)
        for t in range(npc // CK):
            pltpu.sync_copy(rows, c_s.at[pl.ds(sid * npc + t * CK, CK)])
        plsc.subcore_barrier()

        def cb(j, _):
            pltpu.sync_copy(oh_hbm.at[combv.at[j]], rows)
            pltpu.sync_copy(rows, c_s.at[dstv.at[j]], add=True)
            return 0

        lax.fori_loop(0, ECH, cb, 0)
        plsc.subcore_barrier()
        pltpu.sync_copy(c_s.at[pl.ds(sid * npc, npc)],
                        cpart_hbm.at[c, pl.ds(sid * npc, npc)])

    return prep


def _make_aggr(NP, ECH):
    npc = NP // NSUB

    @functools.partial(
        pl.kernel,
        out_type=jax.ShapeDtypeStruct((NC, NP, D), jnp.float32),
        mesh=_MESH,
        scratch_types=[
            pltpu.VMEM((ECH, CK), jnp.int32),
            pltpu.VMEM((ECH, CK), jnp.int32),
            pltpu.VMEM((CK, D), jnp.float32),
            pltpu.VMEM_SHARED((NP, D), jnp.float32),
        ],
    )
    def aggr(h_hbm, src_hbm, dst_hbm, out_hbm, srcv, dstv, rows, acc_s):
        c = lax.axis_index("c")
        sid = lax.axis_index("s")
        wid = _wid()
        pltpu.sync_copy(src_hbm.at[wid], srcv)
        pltpu.sync_copy(dst_hbm.at[wid], dstv)

        _zero_vmem(rows, CK, D)
        for t in range(npc // CK):
            pltpu.sync_copy(rows, acc_s.at[pl.ds(sid * npc + t * CK, CK)])
        plsc.subcore_barrier()

        def body(j, _):
            pltpu.sync_copy(h_hbm.at[srcv.at[j]], rows)
            pltpu.sync_copy(rows, acc_s.at[dstv.at[j]], add=True)
            return 0

        lax.fori_loop(0, ECH, body, 0)
        plsc.subcore_barrier()
        pltpu.sync_copy(acc_s.at[pl.ds(sid * npc, npc)],
                        out_hbm.at[c, pl.ds(sid * npc, npc)])

    return aggr


def _make_pool(NP, NCH):
    gpc = G // NSUB

    @functools.partial(
        pl.kernel,
        out_type=[
            jax.ShapeDtypeStruct((NC, G, D), jnp.float32),
            jax.ShapeDtypeStruct((NC, G, D), jnp.float32),
        ],
        mesh=_MESH,
        scratch_types=[
            pltpu.VMEM((NCH, NK), jnp.int32),
            pltpu.VMEM((NK, D), jnp.float32),
            pltpu.VMEM((NK, D), jnp.float32),
            pltpu.VMEM_SHARED((G, D), jnp.float32),
            pltpu.VMEM_SHARED((G, D), jnp.float32),
        ],
    )
    def pool(hm_hbm, mr_hbm, seg_hbm, sum_hbm, cnt_hbm,
             segv, hrows, mrows, sum_s, cnt_s):
        c = lax.axis_index("c")
        sid = lax.axis_index("s")
        wid = _wid()
        pltpu.sync_copy(seg_hbm.at[wid], segv)

        _zero_vmem(hrows, NK, D)
        pltpu.sync_copy(hrows.at[pl.ds(0, gpc)], sum_s.at[pl.ds(sid * gpc, gpc)])
        pltpu.sync_copy(hrows.at[pl.ds(0, gpc)], cnt_s.at[pl.ds(sid * gpc, gpc)])
        plsc.subcore_barrier()

        def body(j, _):
            base = wid * (NCH * NK) + j * NK
            pltpu.sync_copy(hm_hbm.at[pl.ds(base, NK)], hrows)
            pltpu.sync_copy(mr_hbm.at[pl.ds(base, NK)], mrows)
            pltpu.sync_copy(hrows, sum_s.at[segv.at[j]], add=True)
            pltpu.sync_copy(mrows, cnt_s.at[segv.at[j]], add=True)
            return 0

        lax.fori_loop(0, NCH, body, 0)
        plsc.subcore_barrier()
        pltpu.sync_copy(sum_s.at[pl.ds(sid * gpc, gpc)],
                        sum_hbm.at[c, pl.ds(sid * gpc, gpc)])
        pltpu.sync_copy(cnt_s.at[pl.ds(sid * gpc, gpc)],
                        cnt_hbm.at[c, pl.ds(sid * gpc, gpc)])

    return pool


_SQ = float(np.sqrt(1.0 + BN_EPS))


def _bdot(a, b):
    return jnp.dot(a.astype(jnp.bfloat16), b.astype(jnp.bfloat16),
                   preferred_element_type=jnp.float32)


def _mlp_body(p0, p1, c0, c1, h, t128, crow, w1, b1, w2, b2, g, bb, o, *, relu_out):
    aggr = p0[...] + p1[...] + h[...] + crow[...]
    aggr = aggr + jnp.dot(c0[...] + c1[...], t128[...],
                          preferred_element_type=jnp.float32,
                          precision=lax.Precision.HIGHEST)
    z = _bdot(aggr, w1[...]) + b1[...]
    z = jnp.maximum(z, 0.0)
    z2 = _bdot(z, w2[...]) + b2[...]
    o2 = (z2 / _SQ) * g[...] + bb[...]
    o[...] = jnp.maximum(o2, 0.0) if relu_out else o2


def _mlp_final_body(p0, p1, c0, c1, h, t128, crow, w1, b1, w2, b2, g, bb, m, hm, mr):
    aggr = p0[...] + p1[...] + h[...] + crow[...]
    aggr = aggr + jnp.dot(c0[...] + c1[...], t128[...],
                          preferred_element_type=jnp.float32,
                          precision=lax.Precision.HIGHEST)
    z = _bdot(aggr, w1[...]) + b1[...]
    z = jnp.maximum(z, 0.0)
    z2 = _bdot(z, w2[...]) + b2[...]
    o2 = (z2 / _SQ) * g[...] + bb[...]
    mk = m[...]
    hm[...] = o2 * mk
    mr[...] = jnp.broadcast_to(mk, mr.shape)


def _mlp_specs(BLK):
    row = lambda i: (i, 0)
    full = lambda i: (0, 0)
    return [
        pl.BlockSpec((BLK, D), row),
        pl.BlockSpec((BLK, D), row),
        pl.BlockSpec((BLK, D), row),
        pl.BlockSpec((BLK, D), row),
        pl.BlockSpec((BLK, D), row),
        pl.BlockSpec((D, D), full),
        pl.BlockSpec((1, D), full),
        pl.BlockSpec((D, 2 * D), full),
        pl.BlockSpec((1, 2 * D), full),
        pl.BlockSpec((2 * D, D), full),
        pl.BlockSpec((1, D), full),
        pl.BlockSpec((1, D), full),
        pl.BlockSpec((1, D), full),
    ]


def _run_mlp(parts, cparts, h, t128, crow, w1, b1, w2, b2, g, bb, *, NP, relu_out):
    BLK = 1024
    return pl.pallas_call(
        functools.partial(_mlp_body, relu_out=relu_out),
        grid=(NP // BLK,),
        in_specs=_mlp_specs(BLK),
        out_specs=pl.BlockSpec((BLK, D), lambda i: (i, 0)),
        out_shape=jax.ShapeDtypeStruct((NP, D), jnp.float32),
    )(parts[0], parts[1], cparts[0], cparts[1], h, t128, crow, w1, b1, w2, b2,
      g, bb)


def _run_mlp_final(parts, cparts, h, t128, crow, w1, b1, w2, b2, g, bb, mask, *, NP):
    BLK = 1024
    row = lambda i: (i, 0)
    return pl.pallas_call(
        _mlp_final_body,
        grid=(NP // BLK,),
        in_specs=_mlp_specs(BLK) + [pl.BlockSpec((BLK, 1), row)],
        out_specs=[
            pl.BlockSpec((BLK, D), row),
            pl.BlockSpec((BLK, D), row),
        ],
        out_shape=[
            jax.ShapeDtypeStruct((NP, D), jnp.float32),
            jax.ShapeDtypeStruct((NP, D), jnp.float32),
        ],
    )(parts[0], parts[1], cparts[0], cparts[1], h, t128, crow, w1, b1, w2, b2,
      g, bb, mask)


def _head_body(s0, s1, c0, c1, w1, b1, w2, b2, g, bb, o):
    ssum = s0[...] + s1[...]
    cnt = c0[:, 0:1] + c1[:, 0:1]
    mean = ssum / jnp.maximum(cnt, 1.0)
    pooled = jnp.concatenate([mean, ssum], axis=1)
    z = _bdot(pooled, w1[...]) + b1[...]
    z = jnp.maximum(z, 0.0)
    z = (z / _SQ) * g[...] + bb[...]
    o[...] = _bdot(z, w2[...]) + b2[...]


def _run_head(s0, s1, c0, c1, w1, b1, w2, b2, g, bb):
    return pl.pallas_call(
        _head_body,
        out_shape=jax.ShapeDtypeStruct((G, 8), jnp.float32),
    )(s0, s1, c0, c1, w1, b1, w2, b2, g, bb)


def kernel(x, edge_index, edge_attr, batch, params):
    N = x.shape[0]
    E = edge_index.shape[1]
    NPW = NW * NK
    NP = -(-N // NPW) * NPW
    NCH = NP // NW // NK
    ECH = -(-E // (NW * CK))
    EP = NW * ECH * CK

    src = edge_index[0]
    dst = edge_index[1]
    epad = EP - E
    src_p = jnp.concatenate([src, jnp.zeros((epad,), src.dtype)]).reshape(NW, ECH, CK)
    dst_p = jnp.concatenate([dst, jnp.full((epad,), N, dst.dtype)]).reshape(NW, ECH, CK)
    comb = edge_attr[:, 0] * 3 + edge_attr[:, 1]
    comb_p = jnp.concatenate([comb, jnp.zeros((epad,), comb.dtype)]).reshape(NW, ECH, CK)

    npad = NP - N
    idx0 = x[:, 0] * 4 + x[:, 1]
    idx0_p = jnp.concatenate([idx0, jnp.zeros((npad,), idx0.dtype)]).reshape(NW, NCH, NK)
    seg_p = jnp.concatenate([batch, jnp.zeros((npad,), batch.dtype)]).reshape(NW, NCH, NK)
    mask = (x[:, 2] == 1).astype(jnp.float32)
    mask_p = jnp.concatenate([mask, jnp.zeros((npad,), jnp.float32)])[:, None]

    tab480 = (params['atom_emb1'][:, None, :]
              + params['atom_emb2'][None, :, :]).reshape(-1, D)
    oh128 = jnp.asarray(np.eye(32, D, dtype=np.float32)
                        * (np.arange(32) < 18)[:, None])

    prep = _make_prep(NP, ECH, NCH)
    h, cpart = prep(tab480, oh128, idx0_p, comb_p, dst_p)
    cparts = (cpart[0], cpart[1])

    aggr = _make_aggr(NP, ECH)
    j1 = np.arange(18) // 3
    j2 = np.arange(18) % 3
    num_layer = 5
    for l in range(num_layer):
        w1 = params[f'w1_{l}']
        b1 = params[f'b1_{l}'][None, :]
        w2 = params[f'w2_{l}']
        b2 = params[f'b2_{l}'][None, :]
        g = params[f'bng_{l}'][None, :]
        bb = params[f'bnb_{l}'][None, :]
        t18 = params[f'ee1_{l}'][j1] + params[f'ee2_{l}'][j2]
        t128 = jnp.concatenate([t18, jnp.zeros((D - 18, D), jnp.float32)], axis=0)
        crow = (params[f'ee1_{l}'][4] + params[f'ee2_{l}'][0])[None, :]

        part = aggr(h, src_p, dst_p)
        parts = (part[0], part[1])
        if l < num_layer - 1:
            h = _run_mlp(parts, cparts, h, t128, crow, w1, b1, w2, b2, g, bb,
                         NP=NP, relu_out=True)
        else:
            hm, mrep = _run_mlp_final(parts, cparts, h, t128, crow, w1, b1,
                                      w2, b2, g, bb, mask_p, NP=NP)

    pool = _make_pool(NP, NCH)
    psum, pcnt = pool(hm, mrep, seg_p)

    hw2p = jnp.pad(params['hw2'], ((0, 0), (0, 7)))
    hb2p = jnp.pad(params['hb2'], (0, 7))[None, :]
    out8 = _run_head(psum[0], psum[1], pcnt[0], pcnt[1], params['hw1'],
                     params['hb1'][None, :], hw2p, hb2p,
                     params['hbng'][None, :], params['hbnb'][None, :])
    return out8[:, :1]


# speedup vs baseline: 5.7578x; 2.2378x over previous
import functools

import jax
import jax.numpy as jnp
import numpy as np
from jax import lax
from jax.experimental import pallas as pl
from jax.experimental.pallas import tpu as pltpu
from jax.experimental.pallas import tpu_sc as plsc

D = 128
NW = 32
NC = 2
NSUB = 16
G = 512
CK = 128
NK = 64
BN_EPS = 1e-5

_MESH = plsc.VectorSubcoreMesh(core_axis_name="c", subcore_axis_name="s")


def _wid():
    return lax.axis_index("s") * NC + lax.axis_index("c")


def _zero_vmem(ref, rows, cols):
    zero = jnp.zeros((16,), jnp.float32)

    def zr(r, _):
        def zc(cc, _):
            ref[r, pl.ds(cc * 16, 16)] = zero
            return 0

        return lax.fori_loop(0, cols // 16, zc, 0)

    lax.fori_loop(0, rows, zr, 0)


def _pipelined_scatter(src_tab, idxv, dstv, acc_s, rows2, sem0, sem1, ECH):
    sems = (sem0, sem1)

    def gather(j, b):
        return pltpu.async_copy(src_tab.at[idxv.at[j]], rows2.at[b], sems[b])

    def wait(j, b):
        pltpu.make_async_copy(src_tab.at[pl.ds(0, rows2.shape[1])],
                              rows2.at[b], sems[b]).wait()

    def scat(j, b):
        pltpu.sync_copy(rows2.at[b], acc_s.at[dstv.at[j]], add=True)

    gather(0, 0)

    def body(i, _):
        j0 = 2 * i
        wait(j0, 0)
        gather(j0 + 1, 1)
        scat(j0, 0)
        wait(j0 + 1, 1)
        gather(j0 + 2, 0)
        scat(j0 + 1, 1)
        return 0

    lax.fori_loop(0, (ECH - 1) // 2, body, 0)
    if ECH % 2 == 1:
        wait(ECH - 1, 0)
        scat(ECH - 1, 0)
    else:
        wait(ECH - 2, 0)
        gather(ECH - 1, 1)
        scat(ECH - 2, 0)
        wait(ECH - 1, 1)
        scat(ECH - 1, 1)


def _make_prep(NP, ECH, NCH):
    npc = NP // NSUB

    @functools.partial(
        pl.kernel,
        out_type=[
            jax.ShapeDtypeStruct((NP, D), jnp.float32),
            jax.ShapeDtypeStruct((NC, NP, D), jnp.float32),
        ],
        mesh=_MESH,
        scratch_types=[
            pltpu.VMEM((NCH, NK), jnp.int32),
            pltpu.VMEM(((ECH + 1) // 2, CK), jnp.int32),
            pltpu.VMEM(((ECH + 1) // 2, CK), jnp.int32),
            pltpu.VMEM((2, CK, D), jnp.float32),
            pltpu.VMEM_SHARED((NP, D), jnp.float32),
            pltpu.SemaphoreType.DMA,
            pltpu.SemaphoreType.DMA,
        ],
    )
    def prep(tab_hbm, oh_hbm, idx0_hbm, comb_hbm, dst_hbm, h0_hbm, cpart_hbm,
             idxv, combv, dstv, rows2, c_s, sem0, sem1):
        c = lax.axis_index("c")
        sid = lax.axis_index("s")
        wid = _wid()
        pltpu.sync_copy(idx0_hbm.at[wid], idxv)

        nrows = rows2.at[0, pl.ds(0, NK)]

        def gb(j, _):
            pltpu.sync_copy(tab_hbm.at[idxv.at[j]], nrows)
            pltpu.sync_copy(nrows, h0_hbm.at[pl.ds(wid * (NCH * NK) + j * NK, NK)])
            return 0

        lax.fori_loop(0, NCH, gb, 0)

        rows = rows2.at[0]
        _zero_vmem(rows, CK, D)
        for t in range(npc // CK):
            pltpu.sync_copy(rows, c_s.at[pl.ds(sid * npc + t * CK, CK)])
        plsc.subcore_barrier()

        PH = (ECH + 1) // 2
        for p, n_p in ((0, PH), (1, ECH - PH)):
            pltpu.sync_copy(comb_hbm.at[wid, pl.ds(p * PH, n_p)],
                            combv.at[pl.ds(0, n_p)])
            pltpu.sync_copy(dst_hbm.at[wid, pl.ds(p * PH, n_p)],
                            dstv.at[pl.ds(0, n_p)])
            _pipelined_scatter(oh_hbm, combv, dstv, c_s, rows2, sem0, sem1, n_p)
        plsc.subcore_barrier()
        pltpu.sync_copy(c_s.at[pl.ds(sid * npc, npc)],
                        cpart_hbm.at[c, pl.ds(sid * npc, npc)])

    return prep


def _make_aggr(NP, ECH):
    npc = NP // NSUB

    @functools.partial(
        pl.kernel,
        out_type=jax.ShapeDtypeStruct((NC, NP, D), jnp.float32),
        mesh=_MESH,
        scratch_types=[
            pltpu.VMEM(((ECH + 1) // 2, CK), jnp.int32),
            pltpu.VMEM(((ECH + 1) // 2, CK), jnp.int32),
            pltpu.VMEM((2, CK, D), jnp.float32),
            pltpu.VMEM_SHARED((NP, D), jnp.float32),
            pltpu.SemaphoreType.DMA,
            pltpu.SemaphoreType.DMA,
        ],
    )
    def aggr(h_hbm, src_hbm, dst_hbm, out_hbm, srcv, dstv, rows2, acc_s,
             sem0, sem1):
        c = lax.axis_index("c")
        sid = lax.axis_index("s")
        wid = _wid()

        rows = rows2.at[0]
        _zero_vmem(rows, CK, D)
        for t in range(npc // CK):
            pltpu.sync_copy(rows, acc_s.at[pl.ds(sid * npc + t * CK, CK)])
        plsc.subcore_barrier()

        PH = (ECH + 1) // 2
        for p, n_p in ((0, PH), (1, ECH - PH)):
            pltpu.sync_copy(src_hbm.at[wid, pl.ds(p * PH, n_p)],
                            srcv.at[pl.ds(0, n_p)])
            pltpu.sync_copy(dst_hbm.at[wid, pl.ds(p * PH, n_p)],
                            dstv.at[pl.ds(0, n_p)])
            _pipelined_scatter(h_hbm, srcv, dstv, acc_s, rows2, sem0, sem1, n_p)
        plsc.subcore_barrier()
        pltpu.sync_copy(acc_s.at[pl.ds(sid * npc, npc)],
                        out_hbm.at[c, pl.ds(sid * npc, npc)])

    return aggr


def _make_pool(NP, NCH):
    gpc = G // NSUB

    @functools.partial(
        pl.kernel,
        out_type=[
            jax.ShapeDtypeStruct((NC, G, D), jnp.float32),
            jax.ShapeDtypeStruct((NC, G, D), jnp.float32),
        ],
        mesh=_MESH,
        scratch_types=[
            pltpu.VMEM((NCH, NK), jnp.int32),
            pltpu.VMEM((NK, D), jnp.float32),
            pltpu.VMEM((NK, D), jnp.float32),
            pltpu.VMEM_SHARED((G, D), jnp.float32),
            pltpu.VMEM_SHARED((G, D), jnp.float32),
        ],
    )
    def pool(hm_hbm, mr_hbm, seg_hbm, sum_hbm, cnt_hbm,
             segv, hrows, mrows, sum_s, cnt_s):
        c = lax.axis_index("c")
        sid = lax.axis_index("s")
        wid = _wid()
        pltpu.sync_copy(seg_hbm.at[wid], segv)

        _zero_vmem(hrows, NK, D)
        pltpu.sync_copy(hrows.at[pl.ds(0, gpc)], sum_s.at[pl.ds(sid * gpc, gpc)])
        pltpu.sync_copy(hrows.at[pl.ds(0, gpc)], cnt_s.at[pl.ds(sid * gpc, gpc)])
        plsc.subcore_barrier()

        def body(j, _):
            base = wid * (NCH * NK) + j * NK
            pltpu.sync_copy(hm_hbm.at[pl.ds(base, NK)], hrows)
            pltpu.sync_copy(mr_hbm.at[pl.ds(base, NK)], mrows)
            pltpu.sync_copy(hrows, sum_s.at[segv.at[j]], add=True)
            pltpu.sync_copy(mrows, cnt_s.at[segv.at[j]], add=True)
            return 0

        lax.fori_loop(0, NCH, body, 0)
        plsc.subcore_barrier()
        pltpu.sync_copy(sum_s.at[pl.ds(sid * gpc, gpc)],
                        sum_hbm.at[c, pl.ds(sid * gpc, gpc)])
        pltpu.sync_copy(cnt_s.at[pl.ds(sid * gpc, gpc)],
                        cnt_hbm.at[c, pl.ds(sid * gpc, gpc)])

    return pool


_SQ = float(np.sqrt(1.0 + BN_EPS))


def _bdot(a, b):
    return jnp.dot(a.astype(jnp.bfloat16), b.astype(jnp.bfloat16),
                   preferred_element_type=jnp.float32)


def _mlp_body(p0, p1, c0, c1, h, t128, crow, w1, b1, w2, b2, g, bb, o, *, relu_out):
    aggr = p0[...] + p1[...] + h[...] + crow[...]
    aggr = aggr + jnp.dot(c0[...] + c1[...], t128[...],
                          preferred_element_type=jnp.float32,
                          precision=lax.Precision.HIGHEST)
    z = _bdot(aggr, w1[...]) + b1[...]
    z = jnp.maximum(z, 0.0)
    z2 = _bdot(z, w2[...]) + b2[...]
    o2 = (z2 / _SQ) * g[...] + bb[...]
    o[...] = jnp.maximum(o2, 0.0) if relu_out else o2


def _mlp_final_body(p0, p1, c0, c1, h, t128, crow, w1, b1, w2, b2, g, bb, m, hm, mr):
    aggr = p0[...] + p1[...] + h[...] + crow[...]
    aggr = aggr + jnp.dot(c0[...] + c1[...], t128[...],
                          preferred_element_type=jnp.float32,
                          precision=lax.Precision.HIGHEST)
    z = _bdot(aggr, w1[...]) + b1[...]
    z = jnp.maximum(z, 0.0)
    z2 = _bdot(z, w2[...]) + b2[...]
    o2 = (z2 / _SQ) * g[...] + bb[...]
    mk = m[...]
    hm[...] = o2 * mk
    mr[...] = jnp.broadcast_to(mk, mr.shape)


def _mlp_specs(BLK):
    row = lambda i: (i, 0)
    full = lambda i: (0, 0)
    return [
        pl.BlockSpec((BLK, D), row),
        pl.BlockSpec((BLK, D), row),
        pl.BlockSpec((BLK, D), row),
        pl.BlockSpec((BLK, D), row),
        pl.BlockSpec((BLK, D), row),
        pl.BlockSpec((D, D), full),
        pl.BlockSpec((1, D), full),
        pl.BlockSpec((D, 2 * D), full),
        pl.BlockSpec((1, 2 * D), full),
        pl.BlockSpec((2 * D, D), full),
        pl.BlockSpec((1, D), full),
        pl.BlockSpec((1, D), full),
        pl.BlockSpec((1, D), full),
    ]


def _run_mlp(parts, cparts, h, t128, crow, w1, b1, w2, b2, g, bb, *, NP, relu_out):
    BLK = 1024
    return pl.pallas_call(
        functools.partial(_mlp_body, relu_out=relu_out),
        grid=(NP // BLK,),
        in_specs=_mlp_specs(BLK),
        out_specs=pl.BlockSpec((BLK, D), lambda i: (i, 0)),
        out_shape=jax.ShapeDtypeStruct((NP, D), jnp.float32),
    )(parts[0], parts[1], cparts[0], cparts[1], h, t128, crow, w1, b1, w2, b2,
      g, bb)


def _run_mlp_final(parts, cparts, h, t128, crow, w1, b1, w2, b2, g, bb, mask, *, NP):
    BLK = 1024
    row = lambda i: (i, 0)
    return pl.pallas_call(
        _mlp_final_body,
        grid=(NP // BLK,),
        in_specs=_mlp_specs(BLK) + [pl.BlockSpec((BLK, 1), row)],
        out_specs=[
            pl.BlockSpec((BLK, D), row),
            pl.BlockSpec((BLK, D), row),
        ],
        out_shape=[
            jax.ShapeDtypeStruct((NP, D), jnp.float32),
            jax.ShapeDtypeStruct((NP, D), jnp.float32),
        ],
    )(parts[0], parts[1], cparts[0], cparts[1], h, t128, crow, w1, b1, w2, b2,
      g, bb, mask)


def _head_body(s0, s1, c0, c1, w1, b1, w2, b2, g, bb, o):
    ssum = s0[...] + s1[...]
    cnt = c0[:, 0:1] + c1[:, 0:1]
    mean = ssum / jnp.maximum(cnt, 1.0)
    pooled = jnp.concatenate([mean, ssum], axis=1)
    z = _bdot(pooled, w1[...]) + b1[...]
    z = jnp.maximum(z, 0.0)
    z = (z / _SQ) * g[...] + bb[...]
    o[...] = _bdot(z, w2[...]) + b2[...]


def _run_head(s0, s1, c0, c1, w1, b1, w2, b2, g, bb):
    return pl.pallas_call(
        _head_body,
        out_shape=jax.ShapeDtypeStruct((G, 8), jnp.float32),
    )(s0, s1, c0, c1, w1, b1, w2, b2, g, bb)


def kernel(x, edge_index, edge_attr, batch, params):
    N = x.shape[0]
    E = edge_index.shape[1]
    NPW = NW * NK
    NP = -(-N // NPW) * NPW
    NCH = NP // NW // NK
    ECH = -(-E // (NW * CK))
    EP = NW * ECH * CK

    src = edge_index[0]
    dst = edge_index[1]
    epad = EP - E
    src_p = jnp.concatenate([src, jnp.zeros((epad,), src.dtype)]).reshape(NW, ECH, CK)
    dst_p = jnp.concatenate([dst, jnp.full((epad,), N, dst.dtype)]).reshape(NW, ECH, CK)
    comb = edge_attr[:, 0] * 3 + edge_attr[:, 1]
    comb_p = jnp.concatenate([comb, jnp.zeros((epad,), comb.dtype)]).reshape(NW, ECH, CK)

    npad = NP - N
    idx0 = x[:, 0] * 4 + x[:, 1]
    idx0_p = jnp.concatenate([idx0, jnp.zeros((npad,), idx0.dtype)]).reshape(NW, NCH, NK)
    seg_p = jnp.concatenate([batch, jnp.zeros((npad,), batch.dtype)]).reshape(NW, NCH, NK)
    mask = (x[:, 2] == 1).astype(jnp.float32)
    mask_p = jnp.concatenate([mask, jnp.zeros((npad,), jnp.float32)])[:, None]

    tab480 = (params['atom_emb1'][:, None, :]
              + params['atom_emb2'][None, :, :]).reshape(-1, D)
    _oh = np.eye(32, D, dtype=np.float32) * (np.arange(32) < 18)[:, None]
    oh128 = jnp.asarray(np.tile(_oh, (16, 1)))
    comb_p = comb_p + 32 * (jnp.arange(EP, dtype=jnp.int32).reshape(
        NW, ECH, CK) % 16)

    prep = _make_prep(NP, ECH, NCH)
    h, cpart = prep(tab480, oh128, idx0_p, comb_p, dst_p)
    cparts = (cpart[0], cpart[1])

    aggr = _make_aggr(NP, ECH)
    j1 = np.arange(18) // 3
    j2 = np.arange(18) % 3
    num_layer = 5
    for l in range(num_layer):
        w1 = params[f'w1_{l}']
        b1 = params[f'b1_{l}'][None, :]
        w2 = params[f'w2_{l}']
        b2 = params[f'b2_{l}'][None, :]
        g = params[f'bng_{l}'][None, :]
        bb = params[f'bnb_{l}'][None, :]
        t18 = params[f'ee1_{l}'][j1] + params[f'ee2_{l}'][j2]
        t128 = jnp.concatenate([t18, jnp.zeros((D - 18, D), jnp.float32)], axis=0)
        crow = (params[f'ee1_{l}'][4] + params[f'ee2_{l}'][0])[None, :]

        part = aggr(h, src_p, dst_p)
        parts = (part[0], part[1])
        if l < num_layer - 1:
            h = _run_mlp(parts, cparts, h, t128, crow, w1, b1, w2, b2, g, bb,
                         NP=NP, relu_out=True)
        else:
            hm, mrep = _run_mlp_final(parts, cparts, h, t128, crow, w1, b1,
                                      w2, b2, g, bb, mask_p, NP=NP)

    pool = _make_pool(NP, NCH)
    psum, pcnt = pool(hm, mrep, seg_p)

    hw2p = jnp.pad(params['hw2'], ((0, 0), (0, 7)))
    hb2p = jnp.pad(params['hb2'], (0, 7))[None, :]
    out8 = _run_head(psum[0], psum[1], pcnt[0], pcnt[1], params['hw1'],
                     params['hb1'][None, :], hw2p, hb2p,
                     params['hbng'][None, :], params['hbnb'][None, :])
    return out8[:, :1]


# speedup vs baseline: 6.8228x; 1.1850x over previous
import functools

import jax
import jax.numpy as jnp
import numpy as np
from jax import lax
from jax.experimental import pallas as pl
from jax.experimental.pallas import tpu as pltpu
from jax.experimental.pallas import tpu_sc as plsc

D = 128
NW = 32
NC = 2
NSUB = 16
G = 512
CK = 128
NK = 64
BN_EPS = 1e-5

_MESH = plsc.VectorSubcoreMesh(core_axis_name="c", subcore_axis_name="s")


def _wid():
    return lax.axis_index("s") * NC + lax.axis_index("c")


def _zero_vmem(ref, rows, cols):
    zero = jnp.zeros((16,), jnp.float32)

    def zr(r, _):
        def zc(cc, _):
            ref[r, pl.ds(cc * 16, 16)] = zero
            return 0

        return lax.fori_loop(0, cols // 16, zc, 0)

    lax.fori_loop(0, rows, zr, 0)


def _pipelined_scatter(src_tab, idxv, dstv, acc_s, rows2, sem0, sem1, ECH):
    sems = (sem0, sem1)

    def gather(j, b):
        return pltpu.async_copy(src_tab.at[idxv.at[j]], rows2.at[b], sems[b])

    def wait(j, b):
        pltpu.make_async_copy(src_tab.at[pl.ds(0, rows2.shape[1])],
                              rows2.at[b], sems[b]).wait()

    def scat(j, b):
        pltpu.sync_copy(rows2.at[b], acc_s.at[dstv.at[j]], add=True)

    gather(0, 0)

    def body(i, _):
        j0 = 2 * i
        wait(j0, 0)
        gather(j0 + 1, 1)
        scat(j0, 0)
        wait(j0 + 1, 1)
        gather(j0 + 2, 0)
        scat(j0 + 1, 1)
        return 0

    lax.fori_loop(0, (ECH - 1) // 2, body, 0)
    if ECH % 2 == 1:
        wait(ECH - 1, 0)
        scat(ECH - 1, 0)
    else:
        wait(ECH - 2, 0)
        gather(ECH - 1, 1)
        scat(ECH - 2, 0)
        wait(ECH - 1, 1)
        scat(ECH - 1, 1)


def _make_prep(NP, ECH, NCH):
    npc = NP // NSUB

    @functools.partial(
        pl.kernel,
        out_type=[
            jax.ShapeDtypeStruct((NP, D), jnp.float32),
            jax.ShapeDtypeStruct((NC, NP, D), jnp.float32),
        ],
        mesh=_MESH,
        scratch_types=[
            pltpu.VMEM((NCH, NK), jnp.int32),
            pltpu.VMEM(((ECH + 1) // 2, CK), jnp.int32),
            pltpu.VMEM(((ECH + 1) // 2, CK), jnp.int32),
            pltpu.VMEM((2, CK, D), jnp.float32),
            pltpu.VMEM_SHARED((NP, D), jnp.float32),
            pltpu.SemaphoreType.DMA,
            pltpu.SemaphoreType.DMA,
        ],
    )
    def prep(tab_hbm, oh_hbm, idx0_hbm, comb_hbm, dst_hbm, h0_hbm, cpart_hbm,
             idxv, combv, dstv, rows2, c_s, sem0, sem1):
        c = lax.axis_index("c")
        sid = lax.axis_index("s")
        wid = _wid()
        pltpu.sync_copy(idx0_hbm.at[wid], idxv)

        nrows = rows2.at[0, pl.ds(0, NK)]

        def gb(j, _):
            pltpu.sync_copy(tab_hbm.at[idxv.at[j]], nrows)
            pltpu.sync_copy(nrows, h0_hbm.at[pl.ds(wid * (NCH * NK) + j * NK, NK)])
            return 0

        lax.fori_loop(0, NCH, gb, 0)

        rows = rows2.at[0]
        _zero_vmem(rows, CK, D)
        for t in range(npc // CK):
            pltpu.sync_copy(rows, c_s.at[pl.ds(sid * npc + t * CK, CK)])
        plsc.subcore_barrier()

        PH = (ECH + 1) // 2
        for p, n_p in ((0, PH), (1, ECH - PH)):
            pltpu.sync_copy(comb_hbm.at[wid, pl.ds(p * PH, n_p)],
                            combv.at[pl.ds(0, n_p)])
            pltpu.sync_copy(dst_hbm.at[wid, pl.ds(p * PH, n_p)],
                            dstv.at[pl.ds(0, n_p)])
            _pipelined_scatter(oh_hbm, combv, dstv, c_s, rows2, sem0, sem1, n_p)
        plsc.subcore_barrier()
        pltpu.sync_copy(c_s.at[pl.ds(sid * npc, npc)],
                        cpart_hbm.at[c, pl.ds(sid * npc, npc)])

    return prep


def _make_aggr(NP, ECH):
    npc = NP // NSUB

    @functools.partial(
        pl.kernel,
        out_type=jax.ShapeDtypeStruct((NC, NP, D), jnp.float32),
        mesh=_MESH,
        scratch_types=[
            pltpu.VMEM(((ECH + 1) // 2, CK), jnp.int32),
            pltpu.VMEM(((ECH + 1) // 2, CK), jnp.int32),
            pltpu.VMEM((2, CK, D), jnp.float32),
            pltpu.VMEM_SHARED((NP, D), jnp.float32),
            pltpu.SemaphoreType.DMA,
            pltpu.SemaphoreType.DMA,
        ],
    )
    def aggr(h_hbm, src_hbm, dst_hbm, out_hbm, srcv, dstv, rows2, acc_s,
             sem0, sem1):
        c = lax.axis_index("c")
        sid = lax.axis_index("s")
        wid = _wid()

        rows = rows2.at[0]
        _zero_vmem(rows, CK, D)
        for t in range(npc // CK):
            pltpu.sync_copy(rows, acc_s.at[pl.ds(sid * npc + t * CK, CK)])
        plsc.subcore_barrier()

        PH = (ECH + 1) // 2
        for p, n_p in ((0, PH), (1, ECH - PH)):
            pltpu.sync_copy(src_hbm.at[wid, pl.ds(p * PH, n_p)],
                            srcv.at[pl.ds(0, n_p)])
            pltpu.sync_copy(dst_hbm.at[wid, pl.ds(p * PH, n_p)],
                            dstv.at[pl.ds(0, n_p)])
            _pipelined_scatter(h_hbm, srcv, dstv, acc_s, rows2, sem0, sem1, n_p)
        plsc.subcore_barrier()
        pltpu.sync_copy(acc_s.at[pl.ds(sid * npc, npc)],
                        out_hbm.at[c, pl.ds(sid * npc, npc)])

    return aggr


def _make_pool(NP, NCH):
    gpc = G // NSUB

    @functools.partial(
        pl.kernel,
        out_type=[
            jax.ShapeDtypeStruct((NC, G, D), jnp.float32),
            jax.ShapeDtypeStruct((NC, G, D), jnp.float32),
        ],
        mesh=_MESH,
        scratch_types=[
            pltpu.VMEM((NCH, NK), jnp.int32),
            pltpu.VMEM((NK, D), jnp.float32),
            pltpu.VMEM((NK, D), jnp.float32),
            pltpu.VMEM_SHARED((G, D), jnp.float32),
            pltpu.VMEM_SHARED((G, D), jnp.float32),
        ],
    )
    def pool(hm_hbm, mr_hbm, seg_hbm, sum_hbm, cnt_hbm,
             segv, hrows, mrows, sum_s, cnt_s):
        c = lax.axis_index("c")
        sid = lax.axis_index("s")
        wid = _wid()
        pltpu.sync_copy(seg_hbm.at[wid], segv)

        _zero_vmem(hrows, NK, D)
        pltpu.sync_copy(hrows.at[pl.ds(0, gpc)], sum_s.at[pl.ds(sid * gpc, gpc)])
        pltpu.sync_copy(hrows.at[pl.ds(0, gpc)], cnt_s.at[pl.ds(sid * gpc, gpc)])
        plsc.subcore_barrier()

        def body(j, _):
            base = wid * (NCH * NK) + j * NK
            pltpu.sync_copy(hm_hbm.at[pl.ds(base, NK)], hrows)
            pltpu.sync_copy(mr_hbm.at[pl.ds(base, NK)], mrows)
            pltpu.sync_copy(hrows, sum_s.at[segv.at[j]], add=True)
            pltpu.sync_copy(mrows, cnt_s.at[segv.at[j]], add=True)
            return 0

        lax.fori_loop(0, NCH, body, 0)
        plsc.subcore_barrier()
        pltpu.sync_copy(sum_s.at[pl.ds(sid * gpc, gpc)],
                        sum_hbm.at[c, pl.ds(sid * gpc, gpc)])
        pltpu.sync_copy(cnt_s.at[pl.ds(sid * gpc, gpc)],
                        cnt_hbm.at[c, pl.ds(sid * gpc, gpc)])

    return pool


_SQ = float(np.sqrt(1.0 + BN_EPS))


def _bdot(a, b):
    return jnp.dot(a.astype(jnp.bfloat16), b.astype(jnp.bfloat16),
                   preferred_element_type=jnp.float32)


def _mlp_body(p0, p1, c0, c1, h, t128, crow, w1, b1, w2, b2, g, bb, o, *, relu_out):
    aggr = p0[...] + p1[...] + h[...] + crow[...]
    aggr = aggr + jnp.dot(c0[...] + c1[...], t128[...],
                          preferred_element_type=jnp.float32,
                          precision=lax.Precision.HIGHEST)
    z = _bdot(aggr, w1[...]) + b1[...]
    z = jnp.maximum(z, 0.0)
    z2 = _bdot(z, w2[...]) + b2[...]
    o2 = (z2 / _SQ) * g[...] + bb[...]
    o[...] = jnp.maximum(o2, 0.0) if relu_out else o2


def _mlp_final_body(p0, p1, c0, c1, h, t128, crow, w1, b1, w2, b2, g, bb, m, hm, mr):
    aggr = p0[...] + p1[...] + h[...] + crow[...]
    aggr = aggr + jnp.dot(c0[...] + c1[...], t128[...],
                          preferred_element_type=jnp.float32,
                          precision=lax.Precision.HIGHEST)
    z = _bdot(aggr, w1[...]) + b1[...]
    z = jnp.maximum(z, 0.0)
    z2 = _bdot(z, w2[...]) + b2[...]
    o2 = (z2 / _SQ) * g[...] + bb[...]
    mk = m[...]
    hm[...] = o2 * mk
    mr[...] = jnp.broadcast_to(mk, mr.shape)


def _mlp_specs(BLK):
    row = lambda i: (i, 0)
    full = lambda i: (0, 0)
    return [
        pl.BlockSpec((BLK, D), row),
        pl.BlockSpec((BLK, D), row),
        pl.BlockSpec((BLK, D), row),
        pl.BlockSpec((BLK, D), row),
        pl.BlockSpec((BLK, D), row),
        pl.BlockSpec((D, D), full),
        pl.BlockSpec((1, D), full),
        pl.BlockSpec((D, 2 * D), full),
        pl.BlockSpec((1, 2 * D), full),
        pl.BlockSpec((2 * D, D), full),
        pl.BlockSpec((1, D), full),
        pl.BlockSpec((1, D), full),
        pl.BlockSpec((1, D), full),
    ]


def _run_mlp(parts, cparts, h, t128, crow, w1, b1, w2, b2, g, bb, *, NP, relu_out):
    BLK = 1024
    return pl.pallas_call(
        functools.partial(_mlp_body, relu_out=relu_out),
        grid=(NP // BLK,),
        in_specs=_mlp_specs(BLK),
        out_specs=pl.BlockSpec((BLK, D), lambda i: (i, 0)),
        out_shape=jax.ShapeDtypeStruct((NP, D), jnp.float32),
    )(parts[0], parts[1], cparts[0], cparts[1], h, t128, crow, w1, b1, w2, b2,
      g, bb)


def _run_mlp_final(parts, cparts, h, t128, crow, w1, b1, w2, b2, g, bb, mask, *, NP):
    BLK = 1024
    row = lambda i: (i, 0)
    return pl.pallas_call(
        _mlp_final_body,
        grid=(NP // BLK,),
        in_specs=_mlp_specs(BLK) + [pl.BlockSpec((BLK, 1), row)],
        out_specs=[
            pl.BlockSpec((BLK, D), row),
            pl.BlockSpec((BLK, D), row),
        ],
        out_shape=[
            jax.ShapeDtypeStruct((NP, D), jnp.float32),
            jax.ShapeDtypeStruct((NP, D), jnp.float32),
        ],
    )(parts[0], parts[1], cparts[0], cparts[1], h, t128, crow, w1, b1, w2, b2,
      g, bb, mask)


def _head_body(s0, s1, c0, c1, w1, b1, w2, b2, g, bb, o):
    ssum = s0[...] + s1[...]
    cnt = c0[:, 0:1] + c1[:, 0:1]
    mean = ssum / jnp.maximum(cnt, 1.0)
    pooled = jnp.concatenate([mean, ssum], axis=1)
    z = _bdot(pooled, w1[...]) + b1[...]
    z = jnp.maximum(z, 0.0)
    z = (z / _SQ) * g[...] + bb[...]
    o[...] = _bdot(z, w2[...]) + b2[...]


def _run_head(s0, s1, c0, c1, w1, b1, w2, b2, g, bb):
    return pl.pallas_call(
        _head_body,
        out_shape=jax.ShapeDtypeStruct((G, 8), jnp.float32),
    )(s0, s1, c0, c1, w1, b1, w2, b2, g, bb)


def kernel(x, edge_index, edge_attr, batch, params):
    N = x.shape[0]
    E = edge_index.shape[1]
    NPW = NW * NK
    NP = -(-N // NPW) * NPW
    NCH = NP // NW // NK
    ECH = -(-E // (NW * CK))
    EP = NW * ECH * CK

    src = edge_index[0]
    dst = edge_index[1]
    epad = EP - E
    src_p = jnp.concatenate([src, jnp.zeros((epad,), src.dtype)]).reshape(NW, ECH, CK)
    dummy = N + jnp.arange(epad, dtype=dst.dtype) % (NP - N)
    dst_p = jnp.concatenate([dst, dummy]).reshape(NW, ECH, CK)
    comb = edge_attr[:, 0] * 3 + edge_attr[:, 1]
    comb_p = jnp.concatenate([comb, jnp.zeros((epad,), comb.dtype)]).reshape(NW, ECH, CK)

    npad = NP - N
    idx0 = x[:, 0] * 4 + x[:, 1]
    idx0_p = jnp.concatenate([idx0, jnp.zeros((npad,), idx0.dtype)]).reshape(NW, NCH, NK)
    seg_p = jnp.concatenate([batch, jnp.zeros((npad,), batch.dtype)]).reshape(NW, NCH, NK)
    mask = (x[:, 2] == 1).astype(jnp.float32)
    mask_p = jnp.concatenate([mask, jnp.zeros((npad,), jnp.float32)])[:, None]

    tab480 = (params['atom_emb1'][:, None, :]
              + params['atom_emb2'][None, :, :]).reshape(-1, D)
    _oh = np.eye(32, D, dtype=np.float32) * (np.arange(32) < 18)[:, None]
    oh128 = jnp.asarray(np.tile(_oh, (64, 1)))
    comb_p = comb_p + 32 * (jnp.arange(EP, dtype=jnp.int32).reshape(
        NW, ECH, CK) % 64)

    prep = _make_prep(NP, ECH, NCH)
    h, cpart = prep(tab480, oh128, idx0_p, comb_p, dst_p)
    cparts = (cpart[0], cpart[1])

    aggr = _make_aggr(NP, ECH)
    j1 = np.arange(18) // 3
    j2 = np.arange(18) % 3
    num_layer = 5
    for l in range(num_layer):
        w1 = params[f'w1_{l}']
        b1 = params[f'b1_{l}'][None, :]
        w2 = params[f'w2_{l}']
        b2 = params[f'b2_{l}'][None, :]
        g = params[f'bng_{l}'][None, :]
        bb = params[f'bnb_{l}'][None, :]
        t18 = params[f'ee1_{l}'][j1] + params[f'ee2_{l}'][j2]
        t128 = jnp.concatenate([t18, jnp.zeros((D - 18, D), jnp.float32)], axis=0)
        crow = (params[f'ee1_{l}'][4] + params[f'ee2_{l}'][0])[None, :]

        part = aggr(h, src_p, dst_p)
        parts = (part[0], part[1])
        if l < num_layer - 1:
            h = _run_mlp(parts, cparts, h, t128, crow, w1, b1, w2, b2, g, bb,
                         NP=NP, relu_out=True)
        else:
            hm, mrep = _run_mlp_final(parts, cparts, h, t128, crow, w1, b1,
                                      w2, b2, g, bb, mask_p, NP=NP)

    pool = _make_pool(NP, NCH)
    psum, pcnt = pool(hm, mrep, seg_p)

    hw2p = jnp.pad(params['hw2'], ((0, 0), (0, 7)))
    hb2p = jnp.pad(params['hb2'], (0, 7))[None, :]
    out8 = _run_head(psum[0], psum[1], pcnt[0], pcnt[1], params['hw1'],
                     params['hb1'][None, :], hw2p, hb2p,
                     params['hbng'][None, :], params['hbnb'][None, :])
    return out8[:, :1]
